# baseline (speedup 1.0000x reference)
import functools

import jax
import jax.numpy as jnp
from jax import lax
from jax.experimental import pallas as pl
from jax.experimental.pallas import tpu as pltpu, tpu_sc as plsc

N_ROWS = 50000
D = 512
NSEG = 1024
EPS = 1e-6

NC = 2
NS = 16
NW = NC * NS
RPT = 1568
NPAD = NW * RPT
NXTRA = NPAD + 128
SPT = 40
SROWS = NW * SPT
B1 = 64
B2 = 48
CH = 8

_MESH = plsc.VectorSubcoreMesh(
    core_axis_name="c", subcore_axis_name="s", num_cores=NC, num_subcores=NS)

_LANE = 16
_NCOL = D // _LANE


@functools.partial(
    pl.kernel,
    out_type=(
        jax.ShapeDtypeStruct((SROWS, D), jnp.float32),
        jax.ShapeDtypeStruct((SROWS, D), jnp.float32),
        jax.ShapeDtypeStruct((SROWS, _LANE), jnp.float32),
    ),
    mesh=_MESH,
    scratch_types=[
        pltpu.VMEM((B1, D), jnp.float32),
        pltpu.VMEM((NXTRA + _LANE,), jnp.int32),
        pltpu.VMEM((SPT, D), jnp.float32),
        pltpu.VMEM((SPT, D), jnp.float32),
        pltpu.VMEM((SPT, _LANE), jnp.float32),
    ],
)
def _pass1(x_hbm, ids_hbm, sums_out, sq_out, cnt_out,
           xblk, idsv, loc_s, loc_q, loc_c):
    cid = lax.axis_index("c")
    sid = lax.axis_index("s")
    wid = cid * NS + sid
    lo = wid * SPT
    hi = lo + SPT

    pltpu.sync_copy(ids_hbm, idsv.at[pl.ds(0, NXTRA)])

    z = jnp.zeros((_LANE,), jnp.float32)
    for t in range(SPT):
        for j in range(_NCOL):
            sl = pl.ds(j * _LANE, _LANE)
            loc_s[t, sl] = z
            loc_q[t, sl] = z
        loc_c[t, :] = z

    def _searchsorted(val):
        def _step(_, carry):
            l, h = carry
            m = (l + h) // 2
            v = idsv[pl.ds(m, _LANE)][0]
            go_right = v < val
            return jnp.where(go_right, m + 1, l), jnp.where(go_right, h, m)

        l, _h = lax.fori_loop(0, 17, _step, (jnp.int32(0), jnp.int32(NXTRA)))
        return l

    start = _searchsorted(lo)
    end = _searchsorted(hi)
    start8 = pl.multiple_of((start // 8) * 8, 8)
    nfull = jnp.maximum((end - start8) // B1, 0)
    tstart = start8 + nfull * B1

    def _chunk(rb, rbl, bend):
        v = idsv[pl.ds(rb, _LANE)]
        seg0 = v[0]
        seg7 = v[CH - 1]
        fast = ((seg0 == seg7) & (seg0 >= lo) & (seg0 < hi)
                & (rb + CH <= bend))

        @pl.when(fast)
        def _():
            t = seg0 - lo
            for j in range(_NCOL):
                sl = pl.ds(j * _LANE, _LANE)
                xs = [xblk[rbl + k, sl] for k in range(CH)]
                s01 = (xs[0] + xs[1]) + (xs[2] + xs[3])
                s23 = (xs[4] + xs[5]) + (xs[6] + xs[7])
                plsc.addupdate(loc_s.at[t, sl], s01 + s23)
                q01 = (xs[0] * xs[0] + xs[1] * xs[1]) + \
                      (xs[2] * xs[2] + xs[3] * xs[3])
                q23 = (xs[4] * xs[4] + xs[5] * xs[5]) + \
                      (xs[6] * xs[6] + xs[7] * xs[7])
                plsc.addupdate(loc_q.at[t, sl], q01 + q23)
            plsc.addupdate(loc_c.at[t, :], z + float(CH))

        slo = jnp.where(fast, rb + CH, rb)
        shi = jnp.minimum(rb + CH, bend)
        roff = rb - rbl

        def _row(r, _):
            seg = idsv[pl.ds(r, _LANE)][0]

            @pl.when((seg >= lo) & (seg < hi))
            def _():
                t = seg - lo
                rl = r - roff
                for j in range(_NCOL):
                    sl = pl.ds(j * _LANE, _LANE)
                    xv = xblk[rl, sl]
                    plsc.addupdate(loc_s.at[t, sl], xv)
                    plsc.addupdate(loc_q.at[t, sl], xv * xv)
                plsc.addupdate(loc_c.at[t, :], z + 1.0)
            return 0

        lax.fori_loop(slo, shi, _row, 0)
        return None

    def _block(b, _):
        r0 = pl.multiple_of(start8 + b * B1, 8)
        pltpu.sync_copy(x_hbm.at[pl.ds(r0, B1)], xblk)

        def _c(c, _):
            _chunk(r0 + c * CH, c * CH, r0 + B1)
            return 0

        lax.fori_loop(0, B1 // CH, _c, 0)
        return 0

    lax.fori_loop(0, nfull, _block, 0)

    @pl.when(tstart < end)
    def _():
        e8 = ((end + 7) // 8) * 8
        r0t = pl.multiple_of(jnp.maximum(e8 - B1, 0), 8)
        pltpu.sync_copy(x_hbm.at[pl.ds(r0t, B1)], xblk)
        nct = (end - tstart + CH - 1) // CH

        def _c(c, _):
            rb = tstart + c * CH
            _chunk(rb, rb - r0t, end)
            return 0

        lax.fori_loop(0, nct, _c, 0)

    pltpu.sync_copy(loc_s, sums_out.at[pl.ds(lo, SPT)])
    pltpu.sync_copy(loc_q, sq_out.at[pl.ds(lo, SPT)])
    pltpu.sync_copy(loc_c, cnt_out.at[pl.ds(lo, SPT)])


def _mid_body(sums_ref, sq_ref, cnt_ref, bias_ref, ms_ref):
    s = sums_ref[...]
    q = sq_ref[...]
    c = jnp.maximum(cnt_ref[:, 0:1], 1.0)
    b = bias_ref[...]
    mean = s / c
    var = q / c - mean * mean + b * b
    ms_ref[:, :D] = mean - b
    ms_ref[:, D:] = lax.rsqrt(var + EPS)


def _mid(sums, sq, cnt, bias2d):
    return pl.pallas_call(
        _mid_body,
        out_shape=jax.ShapeDtypeStruct((SROWS, 2 * D), jnp.float32),
    )(sums, sq, cnt, bias2d)


@functools.partial(
    pl.kernel,
    out_type=jax.ShapeDtypeStruct((N_ROWS, D), jnp.float32),
    mesh=_MESH,
    scratch_types=[
        pltpu.VMEM((B2, D), jnp.float32),
        pltpu.VMEM((B2, D), jnp.float32),
        pltpu.VMEM((B2, D), jnp.float32),
        pltpu.VMEM((B2, D), jnp.float32),
        pltpu.VMEM((RPT + _LANE,), jnp.int32),
        pltpu.VMEM((1, 2 * D), jnp.float32),
        pltpu.VMEM((_LANE,), jnp.int32),
        pltpu.SemaphoreType.DMA,
        pltpu.SemaphoreType.DMA,
        pltpu.SemaphoreType.DMA,
        pltpu.SemaphoreType.DMA,
    ],
)
def _pass2(x_hbm, ids_hbm, ms_hbm, y_hbm, xb0, xb1, yb0, yb1, idsv,
           msrow, idxb, si0, si1, so0, so1):
    cid = lax.axis_index("c")
    sid = lax.axis_index("s")
    wid = cid * NS + sid
    base = wid * RPT

    pltpu.sync_copy(ids_hbm.at[pl.ds(base, RPT)], idsv.at[pl.ds(0, RPT)])

    cnt = jnp.minimum(jnp.int32(RPT), jnp.int32(N_ROWS) - base)
    nblk = (cnt + B2 - 1) // B2

    def _r0(b):
        return pl.multiple_of(jnp.minimum(b * B2, cnt - B2), 8)

    def _fetch(seg):
        idxb[...] = jnp.full((_LANE,), seg)
        pltpu.sync_copy(ms_hbm.at[idxb.at[pl.ds(0, 1)]], msrow)

    def _prev_id(r):
        p = idsv[pl.ds(jnp.maximum(r - 1, 0), _LANE)][0]
        return jnp.where(r == 0, jnp.int32(-1), p)

    def _compute(xblk, yblk, r0, force0):
        def _chunk(c, _):
            rbl = c * CH
            rb = r0 + rbl
            v = idsv[pl.ds(rb, _LANE)]
            seg0 = v[0]
            seg7 = v[CH - 1]
            fast = seg0 == seg7
            frc = force0 & (rbl == 0)

            @pl.when(fast & (frc | (seg0 != _prev_id(rb))))
            def _():
                _fetch(seg0)

            @pl.when(fast)
            def _():
                for j in range(_NCOL):
                    sl = pl.ds(j * _LANE, _LANE)
                    sl2 = pl.ds(D + j * _LANE, _LANE)
                    mb = msrow[0, sl]
                    sc = msrow[0, sl2]
                    for k in range(CH):
                        yblk[rbl + k, sl] = (xblk[rbl + k, sl] - mb) * sc

            slo = jnp.where(fast, rb + CH, rb)

            def _row(r, _):
                seg = idsv[pl.ds(r, _LANE)][0]

                @pl.when((force0 & (r == r0)) | (seg != _prev_id(r)))
                def _():
                    _fetch(seg)

                rl = r - r0
                for j in range(_NCOL):
                    sl = pl.ds(j * _LANE, _LANE)
                    sl2 = pl.ds(D + j * _LANE, _LANE)
                    yblk[rl, sl] = ((xblk[rl, sl] - msrow[0, sl])
                                    * msrow[0, sl2])
                return 0

            lax.fori_loop(slo, rb + CH, _row, 0)
            return 0

        lax.fori_loop(0, B2 // CH, _chunk, 0)

    bufs = ((xb0, yb0, si0, so0), (xb1, yb1, si1, so1))

    pltpu.async_copy(x_hbm.at[pl.ds(base + _r0(0), B2)], xb0, si0)

    def _pair(g, _):
        for ph in range(2):
            xb, yb, si, so = bufs[ph]
            b = 2 * g + ph

            @pl.when(b < nblk)
            def _():
                r0 = _r0(b)
                pltpu.make_async_copy(
                    x_hbm.at[pl.ds(base + r0, B2)], xb, si).wait()

                @pl.when(b + 1 < nblk)
                def _():
                    xbn, _yn, sin, _sn = bufs[1 - ph]
                    pltpu.async_copy(
                        x_hbm.at[pl.ds(base + _r0(b + 1), B2)], xbn, sin)

                @pl.when(b >= 2)
                def _():
                    pltpu.make_async_copy(
                        yb, y_hbm.at[pl.ds(base + r0, B2)], so).wait()

                _compute(xb, yb, r0, r0 != b * B2)
                pltpu.async_copy(yb, y_hbm.at[pl.ds(base + r0, B2)], so)
        return 0

    lax.fori_loop(0, (nblk + 1) // 2, _pair, 0)

    for ph in range(2):
        xb, yb, _si, so = bufs[ph]

        @pl.when(((nblk - 1) % 2 == ph) | ((nblk - 2) % 2 == ph))
        def _():
            pltpu.make_async_copy(
                yb, y_hbm.at[pl.ds(base + _r0(nblk - 1), B2)], so).wait()


def kernel(inputs, graph_mask, bias):
    seg = graph_mask.astype(jnp.int32)
    ids = jnp.concatenate([
        seg, jnp.full((NXTRA - N_ROWS,), 2047, jnp.int32)])
    sums, sq, cnt = _pass1(inputs, ids)
    ms = _mid(sums, sq, cnt, bias.reshape(1, D))
    return _pass2(inputs, ids, ms)


# speedup vs baseline: 1.0670x; 1.0670x over previous
import functools

import jax
import jax.numpy as jnp
from jax import lax
from jax.experimental import pallas as pl
from jax.experimental.pallas import tpu as pltpu, tpu_sc as plsc

N_ROWS = 50000
D = 512
NSEG = 1024
EPS = 1e-6

NC = 2
NS = 16
NW = NC * NS
RPT = 1568
NPAD = NW * RPT
NXTRA = NPAD + 128
SPT = 40
SROWS = NW * SPT
B1 = 32
B2 = 48
CH = 8

_MESH = plsc.VectorSubcoreMesh(
    core_axis_name="c", subcore_axis_name="s", num_cores=NC, num_subcores=NS)

_LANE = 16
_NCOL = D // _LANE


@functools.partial(
    pl.kernel,
    out_type=(
        jax.ShapeDtypeStruct((SROWS, D), jnp.float32),
        jax.ShapeDtypeStruct((SROWS, D), jnp.float32),
        jax.ShapeDtypeStruct((SROWS, _LANE), jnp.float32),
    ),
    mesh=_MESH,
    scratch_types=[
        pltpu.VMEM((B1, D), jnp.float32),
        pltpu.VMEM((B1, D), jnp.float32),
        pltpu.VMEM((NXTRA + _LANE,), jnp.int32),
        pltpu.VMEM((SPT, D), jnp.float32),
        pltpu.VMEM((SPT, D), jnp.float32),
        pltpu.VMEM((SPT, _LANE), jnp.float32),
        pltpu.SemaphoreType.DMA,
        pltpu.SemaphoreType.DMA,
    ],
)
def _pass1(x_hbm, ids_hbm, sums_out, sq_out, cnt_out,
           xb0, xb1, idsv, loc_s, loc_q, loc_c, si0, si1):
    cid = lax.axis_index("c")
    sid = lax.axis_index("s")
    wid = cid * NS + sid
    lo = wid * SPT
    hi = lo + SPT

    pltpu.sync_copy(ids_hbm, idsv.at[pl.ds(0, NXTRA)])

    z = jnp.zeros((_LANE,), jnp.float32)
    for t in range(SPT):
        for j in range(_NCOL):
            sl = pl.ds(j * _LANE, _LANE)
            loc_s[t, sl] = z
            loc_q[t, sl] = z
        loc_c[t, :] = z

    def _searchsorted(val):
        def _step(_, carry):
            l, h = carry
            m = (l + h) // 2
            v = idsv[pl.ds(m, _LANE)][0]
            go_right = v < val
            return jnp.where(go_right, m + 1, l), jnp.where(go_right, h, m)

        l, _h = lax.fori_loop(0, 17, _step, (jnp.int32(0), jnp.int32(NXTRA)))
        return l

    start = _searchsorted(lo)
    end = _searchsorted(hi)
    start8 = pl.multiple_of((start // 8) * 8, 8)
    nfull = jnp.maximum((end - start8) // B1, 0)
    tstart = start8 + nfull * B1

    def _chunk(xblk, rb, rbl, bend):
        v = idsv[pl.ds(rb, _LANE)]
        seg0 = v[0]
        seg7 = v[CH - 1]
        fast = ((seg0 == seg7) & (seg0 >= lo) & (seg0 < hi)
                & (rb + CH <= bend))

        @pl.when(fast)
        def _():
            t = seg0 - lo
            for j in range(_NCOL):
                sl = pl.ds(j * _LANE, _LANE)
                xs = [xblk[rbl + k, sl] for k in range(CH)]
                s01 = (xs[0] + xs[1]) + (xs[2] + xs[3])
                s23 = (xs[4] + xs[5]) + (xs[6] + xs[7])
                plsc.addupdate(loc_s.at[t, sl], s01 + s23)
                q01 = (xs[0] * xs[0] + xs[1] * xs[1]) + \
                      (xs[2] * xs[2] + xs[3] * xs[3])
                q23 = (xs[4] * xs[4] + xs[5] * xs[5]) + \
                      (xs[6] * xs[6] + xs[7] * xs[7])
                plsc.addupdate(loc_q.at[t, sl], q01 + q23)
            plsc.addupdate(loc_c.at[t, :], z + float(CH))

        slo = jnp.where(fast, rb + CH, rb)
        shi = jnp.minimum(rb + CH, bend)
        roff = rb - rbl

        def _row(r, _):
            seg = idsv[pl.ds(r, _LANE)][0]

            @pl.when((seg >= lo) & (seg < hi))
            def _():
                t = seg - lo
                rl = r - roff
                for j in range(_NCOL):
                    sl = pl.ds(j * _LANE, _LANE)
                    xv = xblk[rl, sl]
                    plsc.addupdate(loc_s.at[t, sl], xv)
                    plsc.addupdate(loc_q.at[t, sl], xv * xv)
                plsc.addupdate(loc_c.at[t, :], z + 1.0)
            return 0

        lax.fori_loop(slo, shi, _row, 0)
        return None

    bufs = ((xb0, si0), (xb1, si1))

    @pl.when(nfull > 0)
    def _():
        pltpu.async_copy(x_hbm.at[pl.ds(start8, B1)], xb0, si0)

    def _pair(g, _):
        for ph in range(2):
            xb, si = bufs[ph]
            b = 2 * g + ph

            @pl.when(b < nfull)
            def _():
                r0 = pl.multiple_of(start8 + b * B1, 8)
                pltpu.make_async_copy(
                    x_hbm.at[pl.ds(r0, B1)], xb, si).wait()

                @pl.when(b + 1 < nfull)
                def _():
                    xbn, sin = bufs[1 - ph]
                    r0n = pl.multiple_of(r0 + B1, 8)
                    pltpu.async_copy(x_hbm.at[pl.ds(r0n, B1)], xbn, sin)

                def _c(c, _):
                    _chunk(xb, r0 + c * CH, c * CH, r0 + B1)
                    return 0

                lax.fori_loop(0, B1 // CH, _c, 0)
        return 0

    lax.fori_loop(0, (nfull + 1) // 2, _pair, 0)

    @pl.when(tstart < end)
    def _():
        e8 = ((end + 7) // 8) * 8
        r0t = pl.multiple_of(jnp.maximum(e8 - B1, 0), 8)
        pltpu.sync_copy(x_hbm.at[pl.ds(r0t, B1)], xb0)
        nct = (end - tstart + CH - 1) // CH

        def _c(c, _):
            rb = tstart + c * CH
            _chunk(xb0, rb, rb - r0t, end)
            return 0

        lax.fori_loop(0, nct, _c, 0)

    pltpu.sync_copy(loc_s, sums_out.at[pl.ds(lo, SPT)])
    pltpu.sync_copy(loc_q, sq_out.at[pl.ds(lo, SPT)])
    pltpu.sync_copy(loc_c, cnt_out.at[pl.ds(lo, SPT)])


def _mid_body(sums_ref, sq_ref, cnt_ref, bias_ref, ms_ref):
    s = sums_ref[...]
    q = sq_ref[...]
    c = jnp.maximum(cnt_ref[:, 0:1], 1.0)
    b = bias_ref[...]
    mean = s / c
    var = q / c - mean * mean + b * b
    ms_ref[:, :D] = mean - b
    ms_ref[:, D:] = lax.rsqrt(var + EPS)


def _mid(sums, sq, cnt, bias2d):
    return pl.pallas_call(
        _mid_body,
        out_shape=jax.ShapeDtypeStruct((SROWS, 2 * D), jnp.float32),
    )(sums, sq, cnt, bias2d)


@functools.partial(
    pl.kernel,
    out_type=jax.ShapeDtypeStruct((N_ROWS, D), jnp.float32),
    mesh=_MESH,
    scratch_types=[
        pltpu.VMEM((B2, D), jnp.float32),
        pltpu.VMEM((B2, D), jnp.float32),
        pltpu.VMEM((B2, D), jnp.float32),
        pltpu.VMEM((B2, D), jnp.float32),
        pltpu.VMEM((RPT + _LANE,), jnp.int32),
        pltpu.VMEM((1, 2 * D), jnp.float32),
        pltpu.VMEM((_LANE,), jnp.int32),
        pltpu.SemaphoreType.DMA,
        pltpu.SemaphoreType.DMA,
        pltpu.SemaphoreType.DMA,
        pltpu.SemaphoreType.DMA,
    ],
)
def _pass2(x_hbm, ids_hbm, ms_hbm, y_hbm, xb0, xb1, yb0, yb1, idsv,
           msrow, idxb, si0, si1, so0, so1):
    cid = lax.axis_index("c")
    sid = lax.axis_index("s")
    wid = cid * NS + sid
    base = wid * RPT

    pltpu.sync_copy(ids_hbm.at[pl.ds(base, RPT)], idsv.at[pl.ds(0, RPT)])

    cnt = jnp.minimum(jnp.int32(RPT), jnp.int32(N_ROWS) - base)
    nblk = (cnt + B2 - 1) // B2

    def _r0(b):
        return pl.multiple_of(jnp.minimum(b * B2, cnt - B2), 8)

    def _fetch(seg):
        idxb[...] = jnp.full((_LANE,), seg)
        pltpu.sync_copy(ms_hbm.at[idxb.at[pl.ds(0, 1)]], msrow)

    def _prev_id(r):
        p = idsv[pl.ds(jnp.maximum(r - 1, 0), _LANE)][0]
        return jnp.where(r == 0, jnp.int32(-1), p)

    def _compute(xblk, yblk, r0, force0):
        def _chunk(c, _):
            rbl = c * CH
            rb = r0 + rbl
            v = idsv[pl.ds(rb, _LANE)]
            seg0 = v[0]
            seg7 = v[CH - 1]
            fast = seg0 == seg7
            frc = force0 & (rbl == 0)

            @pl.when(fast & (frc | (seg0 != _prev_id(rb))))
            def _():
                _fetch(seg0)

            @pl.when(fast)
            def _():
                for j in range(_NCOL):
                    sl = pl.ds(j * _LANE, _LANE)
                    sl2 = pl.ds(D + j * _LANE, _LANE)
                    mb = msrow[0, sl]
                    sc = msrow[0, sl2]
                    for k in range(CH):
                        yblk[rbl + k, sl] = (xblk[rbl + k, sl] - mb) * sc

            slo = jnp.where(fast, rb + CH, rb)

            def _row(r, _):
                seg = idsv[pl.ds(r, _LANE)][0]

                @pl.when((force0 & (r == r0)) | (seg != _prev_id(r)))
                def _():
                    _fetch(seg)

                rl = r - r0
                for j in range(_NCOL):
                    sl = pl.ds(j * _LANE, _LANE)
                    sl2 = pl.ds(D + j * _LANE, _LANE)
                    yblk[rl, sl] = ((xblk[rl, sl] - msrow[0, sl])
                                    * msrow[0, sl2])
                return 0

            lax.fori_loop(slo, rb + CH, _row, 0)
            return 0

        lax.fori_loop(0, B2 // CH, _chunk, 0)

    bufs = ((xb0, yb0, si0, so0), (xb1, yb1, si1, so1))

    pltpu.async_copy(x_hbm.at[pl.ds(base + _r0(0), B2)], xb0, si0)

    def _pair(g, _):
        for ph in range(2):
            xb, yb, si, so = bufs[ph]
            b = 2 * g + ph

            @pl.when(b < nblk)
            def _():
                r0 = _r0(b)
                pltpu.make_async_copy(
                    x_hbm.at[pl.ds(base + r0, B2)], xb, si).wait()

                @pl.when(b + 1 < nblk)
                def _():
                    xbn, _yn, sin, _sn = bufs[1 - ph]
                    pltpu.async_copy(
                        x_hbm.at[pl.ds(base + _r0(b + 1), B2)], xbn, sin)

                @pl.when(b >= 2)
                def _():
                    pltpu.make_async_copy(
                        yb, y_hbm.at[pl.ds(base + r0, B2)], so).wait()

                _compute(xb, yb, r0, r0 != b * B2)
                pltpu.async_copy(yb, y_hbm.at[pl.ds(base + r0, B2)], so)
        return 0

    lax.fori_loop(0, (nblk + 1) // 2, _pair, 0)

    for ph in range(2):
        xb, yb, _si, so = bufs[ph]

        @pl.when(((nblk - 1) % 2 == ph) | ((nblk - 2) % 2 == ph))
        def _():
            pltpu.make_async_copy(
                yb, y_hbm.at[pl.ds(base + _r0(nblk - 1), B2)], so).wait()


def kernel(inputs, graph_mask, bias):
    seg = graph_mask.astype(jnp.int32)
    ids = jnp.concatenate([
        seg, jnp.full((NXTRA - N_ROWS,), 2047, jnp.int32)])
    sums, sq, cnt = _pass1(inputs, ids)
    ms = _mid(sums, sq, cnt, bias.reshape(1, D))
    return _pass2(inputs, ids, ms)


# speedup vs baseline: 1.0674x; 1.0003x over previous
import functools

import jax
import jax.numpy as jnp
from jax import lax
from jax.experimental import pallas as pl
from jax.experimental.pallas import tpu as pltpu, tpu_sc as plsc

N_ROWS = 50000
D = 512
NSEG = 1024
EPS = 1e-6

NC = 2
NS = 16
NW = NC * NS
RPT = 1568
NPAD = NW * RPT
NXTRA = NPAD + 128
SPT = 40
SROWS = NW * SPT
B1 = 32
B2 = 56
CH = 8

_MESH = plsc.VectorSubcoreMesh(
    core_axis_name="c", subcore_axis_name="s", num_cores=NC, num_subcores=NS)

_LANE = 16
_NCOL = D // _LANE


@functools.partial(
    pl.kernel,
    out_type=(
        jax.ShapeDtypeStruct((SROWS, D), jnp.float32),
        jax.ShapeDtypeStruct((SROWS, D), jnp.float32),
        jax.ShapeDtypeStruct((SROWS, _LANE), jnp.float32),
    ),
    mesh=_MESH,
    scratch_types=[
        pltpu.VMEM((B1, D), jnp.float32),
        pltpu.VMEM((B1, D), jnp.float32),
        pltpu.VMEM((NXTRA + _LANE,), jnp.int32),
        pltpu.VMEM((SPT, D), jnp.float32),
        pltpu.VMEM((SPT, D), jnp.float32),
        pltpu.VMEM((SPT, _LANE), jnp.float32),
        pltpu.SemaphoreType.DMA,
        pltpu.SemaphoreType.DMA,
    ],
)
def _pass1(x_hbm, ids_hbm, sums_out, sq_out, cnt_out,
           xb0, xb1, idsv, loc_s, loc_q, loc_c, si0, si1):
    cid = lax.axis_index("c")
    sid = lax.axis_index("s")
    wid = cid * NS + sid
    lo = wid * SPT
    hi = lo + SPT

    pltpu.sync_copy(ids_hbm, idsv.at[pl.ds(0, NXTRA)])

    z = jnp.zeros((_LANE,), jnp.float32)
    for t in range(SPT):
        for j in range(_NCOL):
            sl = pl.ds(j * _LANE, _LANE)
            loc_s[t, sl] = z
            loc_q[t, sl] = z
        loc_c[t, :] = z

    def _searchsorted(val):
        def _step(_, carry):
            l, h = carry
            m = (l + h) // 2
            v = idsv[pl.ds(m, _LANE)][0]
            go_right = v < val
            return jnp.where(go_right, m + 1, l), jnp.where(go_right, h, m)

        l, _h = lax.fori_loop(0, 17, _step, (jnp.int32(0), jnp.int32(NXTRA)))
        return l

    start = _searchsorted(lo)
    end = _searchsorted(hi)
    start8 = pl.multiple_of((start // 8) * 8, 8)
    nfull = jnp.maximum((end - start8) // B1, 0)
    tstart = start8 + nfull * B1

    def _chunk(xblk, rb, rbl, bend):
        v = idsv[pl.ds(rb, _LANE)]
        seg0 = v[0]
        seg7 = v[CH - 1]
        fast = ((seg0 == seg7) & (seg0 >= lo) & (seg0 < hi)
                & (rb + CH <= bend))

        @pl.when(fast)
        def _():
            t = seg0 - lo
            for j in range(_NCOL):
                sl = pl.ds(j * _LANE, _LANE)
                xs = [xblk[rbl + k, sl] for k in range(CH)]
                s01 = (xs[0] + xs[1]) + (xs[2] + xs[3])
                s23 = (xs[4] + xs[5]) + (xs[6] + xs[7])
                plsc.addupdate(loc_s.at[t, sl], s01 + s23)
                q01 = (xs[0] * xs[0] + xs[1] * xs[1]) + \
                      (xs[2] * xs[2] + xs[3] * xs[3])
                q23 = (xs[4] * xs[4] + xs[5] * xs[5]) + \
                      (xs[6] * xs[6] + xs[7] * xs[7])
                plsc.addupdate(loc_q.at[t, sl], q01 + q23)
            plsc.addupdate(loc_c.at[t, :], z + float(CH))

        slo = jnp.where(fast, rb + CH, rb)
        shi = jnp.minimum(rb + CH, bend)
        roff = rb - rbl

        def _row(r, _):
            seg = idsv[pl.ds(r, _LANE)][0]

            @pl.when((seg >= lo) & (seg < hi))
            def _():
                t = seg - lo
                rl = r - roff
                for j in range(_NCOL):
                    sl = pl.ds(j * _LANE, _LANE)
                    xv = xblk[rl, sl]
                    plsc.addupdate(loc_s.at[t, sl], xv)
                    plsc.addupdate(loc_q.at[t, sl], xv * xv)
                plsc.addupdate(loc_c.at[t, :], z + 1.0)
            return 0

        lax.fori_loop(slo, shi, _row, 0)
        return None

    bufs = ((xb0, si0), (xb1, si1))

    @pl.when(nfull > 0)
    def _():
        pltpu.async_copy(x_hbm.at[pl.ds(start8, B1)], xb0, si0)

    def _pair(g, _):
        for ph in range(2):
            xb, si = bufs[ph]
            b = 2 * g + ph

            @pl.when(b < nfull)
            def _():
                r0 = pl.multiple_of(start8 + b * B1, 8)
                pltpu.make_async_copy(
                    x_hbm.at[pl.ds(r0, B1)], xb, si).wait()

                @pl.when(b + 1 < nfull)
                def _():
                    xbn, sin = bufs[1 - ph]
                    r0n = pl.multiple_of(r0 + B1, 8)
                    pltpu.async_copy(x_hbm.at[pl.ds(r0n, B1)], xbn, sin)

                def _c(c, _):
                    _chunk(xb, r0 + c * CH, c * CH, r0 + B1)
                    return 0

                lax.fori_loop(0, B1 // CH, _c, 0)
        return 0

    lax.fori_loop(0, (nfull + 1) // 2, _pair, 0)

    @pl.when(tstart < end)
    def _():
        e8 = ((end + 7) // 8) * 8
        r0t = pl.multiple_of(jnp.maximum(e8 - B1, 0), 8)
        pltpu.sync_copy(x_hbm.at[pl.ds(r0t, B1)], xb0)
        nct = (end - tstart + CH - 1) // CH

        def _c(c, _):
            rb = tstart + c * CH
            _chunk(xb0, rb, rb - r0t, end)
            return 0

        lax.fori_loop(0, nct, _c, 0)

    pltpu.sync_copy(loc_s, sums_out.at[pl.ds(lo, SPT)])
    pltpu.sync_copy(loc_q, sq_out.at[pl.ds(lo, SPT)])
    pltpu.sync_copy(loc_c, cnt_out.at[pl.ds(lo, SPT)])


def _mid_body(sums_ref, sq_ref, cnt_ref, bias_ref, ms_ref):
    s = sums_ref[...]
    q = sq_ref[...]
    c = jnp.maximum(cnt_ref[:, 0:1], 1.0)
    b = bias_ref[...]
    mean = s / c
    var = q / c - mean * mean + b * b
    ms_ref[:, :D] = mean - b
    ms_ref[:, D:] = lax.rsqrt(var + EPS)


def _mid(sums, sq, cnt, bias2d):
    return pl.pallas_call(
        _mid_body,
        out_shape=jax.ShapeDtypeStruct((SROWS, 2 * D), jnp.float32),
    )(sums, sq, cnt, bias2d)


@functools.partial(
    pl.kernel,
    out_type=jax.ShapeDtypeStruct((N_ROWS, D), jnp.float32),
    mesh=_MESH,
    scratch_types=[
        pltpu.VMEM((B2, D), jnp.float32),
        pltpu.VMEM((B2, D), jnp.float32),
        pltpu.VMEM((B2, D), jnp.float32),
        pltpu.VMEM((B2, D), jnp.float32),
        pltpu.VMEM((RPT + _LANE,), jnp.int32),
        pltpu.VMEM((1, 2 * D), jnp.float32),
        pltpu.VMEM((_LANE,), jnp.int32),
        pltpu.SemaphoreType.DMA,
        pltpu.SemaphoreType.DMA,
        pltpu.SemaphoreType.DMA,
        pltpu.SemaphoreType.DMA,
    ],
)
def _pass2(x_hbm, ids_hbm, ms_hbm, y_hbm, xb0, xb1, yb0, yb1, idsv,
           msrow, idxb, si0, si1, so0, so1):
    cid = lax.axis_index("c")
    sid = lax.axis_index("s")
    wid = cid * NS + sid
    base = wid * RPT

    pltpu.sync_copy(ids_hbm.at[pl.ds(base, RPT)], idsv.at[pl.ds(0, RPT)])

    cnt = jnp.minimum(jnp.int32(RPT), jnp.int32(N_ROWS) - base)
    nblk = (cnt + B2 - 1) // B2

    def _r0(b):
        return pl.multiple_of(jnp.minimum(b * B2, cnt - B2), 8)

    def _fetch(seg):
        idxb[...] = jnp.full((_LANE,), seg)
        pltpu.sync_copy(ms_hbm.at[idxb.at[pl.ds(0, 1)]], msrow)

    def _prev_id(r):
        p = idsv[pl.ds(jnp.maximum(r - 1, 0), _LANE)][0]
        return jnp.where(r == 0, jnp.int32(-1), p)

    def _compute(xblk, yblk, r0, force0):
        def _chunk(c, _):
            rbl = c * CH
            rb = r0 + rbl
            v = idsv[pl.ds(rb, _LANE)]
            seg0 = v[0]
            seg7 = v[CH - 1]
            fast = seg0 == seg7
            frc = force0 & (rbl == 0)

            @pl.when(fast & (frc | (seg0 != _prev_id(rb))))
            def _():
                _fetch(seg0)

            @pl.when(fast)
            def _():
                for j in range(_NCOL):
                    sl = pl.ds(j * _LANE, _LANE)
                    sl2 = pl.ds(D + j * _LANE, _LANE)
                    mb = msrow[0, sl]
                    sc = msrow[0, sl2]
                    for k in range(CH):
                        yblk[rbl + k, sl] = (xblk[rbl + k, sl] - mb) * sc

            slo = jnp.where(fast, rb + CH, rb)

            def _row(r, _):
                seg = idsv[pl.ds(r, _LANE)][0]

                @pl.when((force0 & (r == r0)) | (seg != _prev_id(r)))
                def _():
                    _fetch(seg)

                rl = r - r0
                for j in range(_NCOL):
                    sl = pl.ds(j * _LANE, _LANE)
                    sl2 = pl.ds(D + j * _LANE, _LANE)
                    yblk[rl, sl] = ((xblk[rl, sl] - msrow[0, sl])
                                    * msrow[0, sl2])
                return 0

            lax.fori_loop(slo, rb + CH, _row, 0)
            return 0

        lax.fori_loop(0, B2 // CH, _chunk, 0)

    bufs = ((xb0, yb0, si0, so0), (xb1, yb1, si1, so1))

    pltpu.async_copy(x_hbm.at[pl.ds(base + _r0(0), B2)], xb0, si0)

    def _pair(g, _):
        for ph in range(2):
            xb, yb, si, so = bufs[ph]
            b = 2 * g + ph

            @pl.when(b < nblk)
            def _():
                r0 = _r0(b)
                pltpu.make_async_copy(
                    x_hbm.at[pl.ds(base + r0, B2)], xb, si).wait()

                @pl.when(b + 1 < nblk)
                def _():
                    xbn, _yn, sin, _sn = bufs[1 - ph]
                    pltpu.async_copy(
                        x_hbm.at[pl.ds(base + _r0(b + 1), B2)], xbn, sin)

                @pl.when(b >= 2)
                def _():
                    pltpu.make_async_copy(
                        yb, y_hbm.at[pl.ds(base + r0, B2)], so).wait()

                _compute(xb, yb, r0, r0 != b * B2)
                pltpu.async_copy(yb, y_hbm.at[pl.ds(base + r0, B2)], so)
        return 0

    lax.fori_loop(0, (nblk + 1) // 2, _pair, 0)

    for ph in range(2):
        xb, yb, _si, so = bufs[ph]

        @pl.when(((nblk - 1) % 2 == ph) | ((nblk - 2) % 2 == ph))
        def _():
            pltpu.make_async_copy(
                yb, y_hbm.at[pl.ds(base + _r0(nblk - 1), B2)], so).wait()


def kernel(inputs, graph_mask, bias):
    seg = graph_mask.astype(jnp.int32)
    ids = jnp.concatenate([
        seg, jnp.full((NXTRA - N_ROWS,), 2047, jnp.int32)])
    sums, sq, cnt = _pass1(inputs, ids)
    ms = _mid(sums, sq, cnt, bias.reshape(1, D))
    return _pass2(inputs, ids, ms)
